# baseline (speedup 1.0000x reference)
import functools

import jax
import jax.numpy as jnp
from jax import lax
from jax.experimental import pallas as pl
from jax.experimental.pallas import tpu as pltpu
from jax.experimental.pallas import tpu_sc as plsc

_D_IN = 64
_D_EMB = 32
_N_EFF = 8
_N_LABELS = 1024
_R = 1024
_NC = 2
_NS = 16
_G = 256
_SCAT = 128
_SW = 48


def _enc_body(x_ref, we_ref, be_ref, wd_ref, bd_ref,
              dec_ref, stats_ref, lab_ref):
    xb = x_ref[...]
    emb = jnp.tanh(
        lax.dot_general(xb, we_ref[...], (((1,), (0,)), ((), ())),
                        preferred_element_type=jnp.float32)
        + be_ref[...])
    dec_ref[...] = (
        lax.dot_general(emb, wd_ref[...], (((1,), (0,)), ((), ())),
                        preferred_element_type=jnp.float32)
        + bd_ref[...])
    cons = emb[:, _N_EFF:]
    stats_ref[...] = jnp.concatenate([cons, cons * cons], axis=1)
    lab_ref[...] = xb[:, 0].astype(jnp.int32).reshape(_R // 128, 128)


def _sc_body(stats_hbm, labels_hbm, zt_hbm, zc_hbm, ones_hbm,
             s_out, c_out, table, ctable, sv0, sv1, labels_v, ones_v,
             lsem0, lsem1, ssem0, ssem1):
    cid = lax.axis_index("c")
    sid = lax.axis_index("s")
    wid = cid * _NS + sid
    rows_per_w = stats_hbm.shape[0] // (_NC * _NS)

    @pl.when(sid == 0)
    def _zero():
        pltpu.sync_copy(zt_hbm, table)
        pltpu.sync_copy(zc_hbm, ctable)

    pltpu.sync_copy(ones_hbm, ones_v)
    pltpu.sync_copy(
        labels_hbm.at[pl.ds(wid * (rows_per_w // 128), rows_per_w // 128)],
        labels_v)
    plsc.subcore_barrier()

    base = wid * rows_per_w
    nchunk = rows_per_w // _G
    ksub = _G // _SCAT
    bufs = (sv0, sv1)
    lsems = (lsem0, lsem1)
    ssems = (ssem0, ssem1)

    def chunk_src(c):
        return stats_hbm.at[pl.ds(base + c * _G, _G)]

    loads = [pltpu.async_copy(chunk_src(0), bufs[0], lsems[0]), None]
    pending = [[], []]
    for c in range(nchunk):
        b = c % 2
        loads[b].wait()
        descs = []
        for k in range(ksub):
            idx = labels_v.at[c * ksub + k]
            descs.append(pltpu.async_copy(
                bufs[b].at[pl.ds(k * _SCAT, _SCAT)], table.at[idx],
                ssems[b], add=True))
            descs.append(pltpu.async_copy(
                ones_v, ctable.at[idx], ssems[b], add=True))
        if c + 1 < nchunk:
            nb = 1 - b
            for d in pending[nb]:
                d.wait()
            pending[nb] = []
            loads[nb] = pltpu.async_copy(chunk_src(c + 1), bufs[nb],
                                         lsems[nb])
        pending[b] = descs
    for b in (0, 1):
        for d in pending[b]:
            d.wait()

    plsc.subcore_barrier()

    @pl.when(sid == 0)
    def _emit():
        pltpu.sync_copy(table, s_out.at[cid])
        pltpu.sync_copy(ctable, c_out.at[cid])


def _fin_body(s_ref, c_ref, res_ref):
    sv = s_ref[0] + s_ref[1]
    cv = c_ref[0] + c_ref[1]
    counts = cv[:, 0:1]
    s = sv[:, :_SW // 2]
    s2 = sv[:, _SW // 2:]
    n = jnp.maximum(counts, 1.0)
    m = s / n
    mse = (s2 - 2.0 * m * s) / n + m * m
    present = counts > 0.0
    n_present = jnp.maximum(jnp.sum(present.astype(jnp.float32)), 1.0)
    res = jnp.sum(jnp.where(present, mse, 0.0)) / (
        n_present * (_D_EMB - _N_EFF))
    res_ref[...] = jnp.full((8, 128), res, jnp.float32)


def kernel(x, W_enc, b_enc, W_dec, b_dec):
    b_rows = x.shape[0]
    grid = b_rows // _R
    we_pad = jnp.concatenate([jnp.zeros((1, _D_EMB), W_enc.dtype), W_enc],
                             axis=0)

    dec, stats, labels2d = pl.pallas_call(
        _enc_body,
        grid=(grid,),
        in_specs=[
            pl.BlockSpec((_R, _D_IN + 1), lambda i: (i, 0)),
            pl.BlockSpec((_D_IN + 1, _D_EMB), lambda i: (0, 0)),
            pl.BlockSpec((1, _D_EMB), lambda i: (0, 0)),
            pl.BlockSpec((_D_EMB, _D_IN), lambda i: (0, 0)),
            pl.BlockSpec((1, _D_IN), lambda i: (0, 0)),
        ],
        out_specs=[
            pl.BlockSpec((_R, _D_IN), lambda i: (i, 0)),
            pl.BlockSpec((_R, _SW), lambda i: (i, 0)),
            pl.BlockSpec((_R // 128, 128), lambda i: (i, 0)),
        ],
        out_shape=[
            jax.ShapeDtypeStruct((b_rows, _D_IN), jnp.float32),
            jax.ShapeDtypeStruct((b_rows, _SW), jnp.float32),
            jax.ShapeDtypeStruct((b_rows // 128, 128), jnp.int32),
        ],
    )(x, we_pad, b_enc.reshape(1, -1), W_dec, b_dec.reshape(1, -1))

    mesh = plsc.VectorSubcoreMesh(core_axis_name="c", subcore_axis_name="s")
    sc_fn = pl.kernel(
        _sc_body,
        out_type=[
            jax.ShapeDtypeStruct((_NC, _N_LABELS, _SW), jnp.float32),
            jax.ShapeDtypeStruct((_NC, _N_LABELS, 16), jnp.float32),
        ],
        mesh=mesh,
        scratch_types=[
            pltpu.VMEM_SHARED((_N_LABELS, _SW), jnp.float32),
            pltpu.VMEM_SHARED((_N_LABELS, 16), jnp.float32),
            pltpu.VMEM((_G, _SW), jnp.float32),
            pltpu.VMEM((_G, _SW), jnp.float32),
            pltpu.VMEM((64, 128), jnp.int32),
            pltpu.VMEM((_SCAT, 16), jnp.float32),
            pltpu.SemaphoreType.DMA,
            pltpu.SemaphoreType.DMA,
            pltpu.SemaphoreType.DMA,
            pltpu.SemaphoreType.DMA,
        ],
    )
    zt = jnp.zeros((_N_LABELS, _SW), jnp.float32)
    zc = jnp.zeros((_N_LABELS, 16), jnp.float32)
    ones = jnp.ones((_SCAT, 16), jnp.float32)
    s_tab, c_tab = sc_fn(stats, labels2d, zt, zc, ones)

    res = pl.pallas_call(
        _fin_body,
        out_shape=jax.ShapeDtypeStruct((8, 128), jnp.float32),
    )(s_tab, c_tab)
    return dec, res[0, 0]


# speedup vs baseline: 1.2257x; 1.2257x over previous
import functools

import jax
import jax.numpy as jnp
from jax import lax
from jax.experimental import pallas as pl
from jax.experimental.pallas import tpu as pltpu
from jax.experimental.pallas import tpu_sc as plsc

_D_IN = 64
_D_EMB = 32
_N_EFF = 8
_N_LABELS = 1024
_R = 2048
_NC = 2
_NS = 16
_G = 256
_SCAT = 128
_SW = 64


def _enc_body(x_ref, we_ref, be_ref, wd_ref, bd_ref,
              dec_ref, stats_ref, lab_ref):
    xb = x_ref[...]
    emb = jnp.tanh(
        lax.dot_general(xb, we_ref[...], (((1,), (0,)), ((), ())),
                        preferred_element_type=jnp.float32)
        + be_ref[...])
    dec_ref[...] = (
        lax.dot_general(emb, wd_ref[...], (((1,), (0,)), ((), ())),
                        preferred_element_type=jnp.float32)
        + bd_ref[...])
    stats_ref[...] = jnp.concatenate([emb, emb * emb], axis=1)
    lab_ref[...] = xb[:, 0].astype(jnp.int32).reshape(_R // 128, 128)


def _sc_body(stats_hbm, labels_hbm, zt_hbm, zc_hbm, ones_hbm,
             s_out, c_out, table, ctable, sv0, sv1, labels_v, ones_v,
             lsem0, lsem1, ssem0, ssem1):
    cid = lax.axis_index("c")
    sid = lax.axis_index("s")
    wid = cid * _NS + sid
    rows_per_w = stats_hbm.shape[0] // (_NC * _NS)

    @pl.when(sid == 0)
    def _zero():
        pltpu.sync_copy(zt_hbm, table)
        pltpu.sync_copy(zc_hbm, ctable)

    pltpu.sync_copy(ones_hbm, ones_v)
    pltpu.sync_copy(
        labels_hbm.at[pl.ds(wid * (rows_per_w // 128), rows_per_w // 128)],
        labels_v)
    plsc.subcore_barrier()

    base = wid * rows_per_w
    nchunk = rows_per_w // _G
    ksub = _G // _SCAT
    bufs = (sv0, sv1)
    lsems = (lsem0, lsem1)
    ssems = (ssem0, ssem1)

    def chunk_src(c):
        return stats_hbm.at[pl.ds(base + c * _G, _G)]

    loads = [pltpu.async_copy(chunk_src(0), bufs[0], lsems[0]), None]
    pending = [[], []]
    for c in range(nchunk):
        b = c % 2
        loads[b].wait()
        descs = []
        for k in range(ksub):
            idx = labels_v.at[c * ksub + k]
            descs.append(pltpu.async_copy(
                bufs[b].at[pl.ds(k * _SCAT, _SCAT)], table.at[idx],
                ssems[b], add=True))
            descs.append(pltpu.async_copy(
                ones_v, ctable.at[idx], ssems[b], add=True))
        if c + 1 < nchunk:
            nb = 1 - b
            for d in pending[nb]:
                d.wait()
            pending[nb] = []
            loads[nb] = pltpu.async_copy(chunk_src(c + 1), bufs[nb],
                                         lsems[nb])
        pending[b] = descs
    for b in (0, 1):
        for d in pending[b]:
            d.wait()

    plsc.subcore_barrier()

    @pl.when(sid == 0)
    def _emit():
        pltpu.sync_copy(table, s_out.at[cid])
        pltpu.sync_copy(ctable, c_out.at[cid])


def _fin_body(s_ref, c_ref, res_ref):
    sv = s_ref[0] + s_ref[1]
    cv = c_ref[0] + c_ref[1]
    counts = cv[:, 0:1]
    s = sv[:, _N_EFF:_D_EMB]
    s2 = sv[:, _D_EMB + _N_EFF:]
    n = jnp.maximum(counts, 1.0)
    m = s / n
    mse = (s2 - 2.0 * m * s) / n + m * m
    present = counts > 0.0
    n_present = jnp.maximum(jnp.sum(present.astype(jnp.float32)), 1.0)
    res = jnp.sum(jnp.where(present, mse, 0.0)) / (
        n_present * (_D_EMB - _N_EFF))
    res_ref[...] = jnp.full((8, 128), res, jnp.float32)


def kernel(x, W_enc, b_enc, W_dec, b_dec):
    b_rows = x.shape[0]
    grid = b_rows // _R
    we_pad = jnp.concatenate([jnp.zeros((1, _D_EMB), W_enc.dtype), W_enc],
                             axis=0)

    dec, stats, labels2d = pl.pallas_call(
        _enc_body,
        grid=(grid,),
        in_specs=[
            pl.BlockSpec((_R, _D_IN + 1), lambda i: (i, 0)),
            pl.BlockSpec((_D_IN + 1, _D_EMB), lambda i: (0, 0)),
            pl.BlockSpec((1, _D_EMB), lambda i: (0, 0)),
            pl.BlockSpec((_D_EMB, _D_IN), lambda i: (0, 0)),
            pl.BlockSpec((1, _D_IN), lambda i: (0, 0)),
        ],
        out_specs=[
            pl.BlockSpec((_R, _D_IN), lambda i: (i, 0)),
            pl.BlockSpec((_R, _SW), lambda i: (i, 0)),
            pl.BlockSpec((_R // 128, 128), lambda i: (i, 0)),
        ],
        out_shape=[
            jax.ShapeDtypeStruct((b_rows, _D_IN), jnp.float32),
            jax.ShapeDtypeStruct((b_rows, _SW), jnp.float32),
            jax.ShapeDtypeStruct((b_rows // 128, 128), jnp.int32),
        ],
    )(x, we_pad, b_enc.reshape(1, -1), W_dec, b_dec.reshape(1, -1))

    mesh = plsc.VectorSubcoreMesh(core_axis_name="c", subcore_axis_name="s")
    sc_fn = pl.kernel(
        _sc_body,
        out_type=[
            jax.ShapeDtypeStruct((_NC, _N_LABELS, _SW), jnp.float32),
            jax.ShapeDtypeStruct((_NC, _N_LABELS, 16), jnp.float32),
        ],
        mesh=mesh,
        scratch_types=[
            pltpu.VMEM_SHARED((_N_LABELS, _SW), jnp.float32),
            pltpu.VMEM_SHARED((_N_LABELS, 16), jnp.float32),
            pltpu.VMEM((_G, _SW), jnp.float32),
            pltpu.VMEM((_G, _SW), jnp.float32),
            pltpu.VMEM((64, 128), jnp.int32),
            pltpu.VMEM((_SCAT, 16), jnp.float32),
            pltpu.SemaphoreType.DMA,
            pltpu.SemaphoreType.DMA,
            pltpu.SemaphoreType.DMA,
            pltpu.SemaphoreType.DMA,
        ],
    )
    zt = jnp.zeros((_N_LABELS, _SW), jnp.float32)
    zc = jnp.zeros((_N_LABELS, 16), jnp.float32)
    ones = jnp.ones((_SCAT, 16), jnp.float32)
    s_tab, c_tab = sc_fn(stats, labels2d, zt, zc, ones)

    res = pl.pallas_call(
        _fin_body,
        out_shape=jax.ShapeDtypeStruct((8, 128), jnp.float32),
    )(s_tab, c_tab)
    return dec, res[0, 0]


# speedup vs baseline: 1.4163x; 1.1555x over previous
import functools

import jax
import jax.numpy as jnp
from jax import lax
from jax.experimental import pallas as pl
from jax.experimental.pallas import tpu as pltpu
from jax.experimental.pallas import tpu_sc as plsc

_D_IN = 64
_D_EMB = 32
_N_EFF = 8
_N_LABELS = 1024
_R = 8192
_NC = 2
_NS = 16
_G = 256
_SCAT = 128
_SW = 64


def _enc_body(x_ref, we_ref, be_ref, wd_ref, bd_ref,
              dec_ref, stats_ref, lab_ref):
    xb = x_ref[...]
    emb = jnp.tanh(
        lax.dot_general(xb, we_ref[...], (((1,), (0,)), ((), ())),
                        preferred_element_type=jnp.float32)
        + be_ref[...])
    dec_ref[...] = (
        lax.dot_general(emb, wd_ref[...], (((1,), (0,)), ((), ())),
                        preferred_element_type=jnp.float32)
        + bd_ref[...])
    stats_ref[...] = jnp.concatenate([emb, emb * emb], axis=1)
    lab_ref[...] = xb[:, 0].astype(jnp.int32).reshape(_R // 128, 128)


def _sc_body(stats_hbm, labels_hbm, zt_hbm, zc_hbm, ones_hbm,
             s_out, c_out, table, ctable, sv0, sv1, labels_v, ones_v,
             lsem0, lsem1, ssem0, ssem1):
    cid = lax.axis_index("c")
    sid = lax.axis_index("s")
    wid = cid * _NS + sid
    rows_per_w = stats_hbm.shape[0] // (_NC * _NS)

    @pl.when(sid == 0)
    def _zero():
        pltpu.sync_copy(zt_hbm, table)
        pltpu.sync_copy(zc_hbm, ctable)

    pltpu.sync_copy(ones_hbm, ones_v)
    pltpu.sync_copy(
        labels_hbm.at[pl.ds(wid * (rows_per_w // 128), rows_per_w // 128)],
        labels_v)
    plsc.subcore_barrier()

    base = wid * rows_per_w
    nchunk = rows_per_w // _G
    ksub = _G // _SCAT
    bufs = (sv0, sv1)
    lsems = (lsem0, lsem1)
    ssems = (ssem0, ssem1)

    def chunk_src(c):
        return stats_hbm.at[pl.ds(base + c * _G, _G)]

    loads = [pltpu.async_copy(chunk_src(0), bufs[0], lsems[0]), None]
    pending = [[], []]
    for c in range(nchunk):
        b = c % 2
        loads[b].wait()
        descs = []
        for k in range(ksub):
            idx = labels_v.at[c * ksub + k]
            descs.append(pltpu.async_copy(
                bufs[b].at[pl.ds(k * _SCAT, _SCAT)], table.at[idx],
                ssems[b], add=True))
            descs.append(pltpu.async_copy(
                ones_v, ctable.at[idx], ssems[b], add=True))
        if c + 1 < nchunk:
            nb = 1 - b
            for d in pending[nb]:
                d.wait()
            pending[nb] = []
            loads[nb] = pltpu.async_copy(chunk_src(c + 1), bufs[nb],
                                         lsems[nb])
        pending[b] = descs
    for b in (0, 1):
        for d in pending[b]:
            d.wait()

    plsc.subcore_barrier()

    @pl.when(sid == 0)
    def _emit():
        pltpu.sync_copy(table, s_out.at[cid])
        pltpu.sync_copy(ctable, c_out.at[cid])


def _fin_body(s_ref, c_ref, res_ref):
    sv = s_ref[0] + s_ref[1]
    cv = c_ref[0] + c_ref[1]
    counts = cv[:, 0:1]
    s = sv[:, _N_EFF:_D_EMB]
    s2 = sv[:, _D_EMB + _N_EFF:]
    n = jnp.maximum(counts, 1.0)
    m = s / n
    mse = (s2 - 2.0 * m * s) / n + m * m
    present = counts > 0.0
    n_present = jnp.maximum(jnp.sum(present.astype(jnp.float32)), 1.0)
    res = jnp.sum(jnp.where(present, mse, 0.0)) / (
        n_present * (_D_EMB - _N_EFF))
    res_ref[...] = jnp.full((8, 128), res, jnp.float32)


def kernel(x, W_enc, b_enc, W_dec, b_dec):
    b_rows = x.shape[0]
    grid = b_rows // _R
    we_pad = jnp.concatenate([jnp.zeros((1, _D_EMB), W_enc.dtype), W_enc],
                             axis=0)

    dec, stats, labels2d = pl.pallas_call(
        _enc_body,
        grid=(grid,),
        in_specs=[
            pl.BlockSpec((_R, _D_IN + 1), lambda i: (i, 0)),
            pl.BlockSpec((_D_IN + 1, _D_EMB), lambda i: (0, 0)),
            pl.BlockSpec((1, _D_EMB), lambda i: (0, 0)),
            pl.BlockSpec((_D_EMB, _D_IN), lambda i: (0, 0)),
            pl.BlockSpec((1, _D_IN), lambda i: (0, 0)),
        ],
        out_specs=[
            pl.BlockSpec((_R, _D_IN), lambda i: (i, 0)),
            pl.BlockSpec((_R, _SW), lambda i: (i, 0)),
            pl.BlockSpec((_R // 128, 128), lambda i: (i, 0)),
        ],
        out_shape=[
            jax.ShapeDtypeStruct((b_rows, _D_IN), jnp.float32),
            jax.ShapeDtypeStruct((b_rows, _SW), jnp.float32),
            jax.ShapeDtypeStruct((b_rows // 128, 128), jnp.int32),
        ],
    )(x, we_pad, b_enc.reshape(1, -1), W_dec, b_dec.reshape(1, -1))

    mesh = plsc.VectorSubcoreMesh(core_axis_name="c", subcore_axis_name="s")
    sc_fn = pl.kernel(
        _sc_body,
        out_type=[
            jax.ShapeDtypeStruct((_NC, _N_LABELS, _SW), jnp.float32),
            jax.ShapeDtypeStruct((_NC, _N_LABELS, 16), jnp.float32),
        ],
        mesh=mesh,
        scratch_types=[
            pltpu.VMEM_SHARED((_N_LABELS, _SW), jnp.float32),
            pltpu.VMEM_SHARED((_N_LABELS, 16), jnp.float32),
            pltpu.VMEM((_G, _SW), jnp.float32),
            pltpu.VMEM((_G, _SW), jnp.float32),
            pltpu.VMEM((64, 128), jnp.int32),
            pltpu.VMEM((_SCAT, 16), jnp.float32),
            pltpu.SemaphoreType.DMA,
            pltpu.SemaphoreType.DMA,
            pltpu.SemaphoreType.DMA,
            pltpu.SemaphoreType.DMA,
        ],
    )
    zt = jnp.zeros((_N_LABELS, _SW), jnp.float32)
    zc = jnp.zeros((_N_LABELS, 16), jnp.float32)
    ones = jnp.ones((_SCAT, 16), jnp.float32)
    s_tab, c_tab = sc_fn(stats, labels2d, zt, zc, ones)

    res = pl.pallas_call(
        _fin_body,
        out_shape=jax.ShapeDtypeStruct((8, 128), jnp.float32),
    )(s_tab, c_tab)
    return dec, res[0, 0]


# speedup vs baseline: 1.4172x; 1.0007x over previous
import jax
import jax.numpy as jnp
from jax import lax
from jax.experimental import pallas as pl
from jax.experimental.pallas import tpu as pltpu
from jax.experimental.pallas import tpu_sc as plsc

_D_IN = 64
_D_EMB = 32
_N_EFF = 8
_N_LABELS = 1024
_R = 8192
_NC = 2
_NS = 16
_G = 256
_SCAT = 128
_SW = 64


def _enc_body(x_ref, we_ref, be_ref, wd_ref, bd_ref,
              dec_ref, stats_ref, lab_ref):
    xb = x_ref[...]
    emb = jnp.tanh(
        lax.dot_general(xb, we_ref[...], (((1,), (0,)), ((), ())),
                        preferred_element_type=jnp.float32)
        + be_ref[...])
    dec_ref[...] = (
        lax.dot_general(emb, wd_ref[...], (((1,), (0,)), ((), ())),
                        preferred_element_type=jnp.float32)
        + bd_ref[...])
    stats_ref[...] = jnp.concatenate([emb, emb * emb], axis=1)
    lab_ref[...] = xb[:, 0].astype(jnp.int32).reshape(_R // 128, 128)


def _sc_body(stats_hbm, labels_hbm, zt_hbm, zc_hbm, ones_hbm,
             s_out, c_out, table, ctable, sv0, sv1, labels_v, ones_v,
             lsem0, lsem1, ssem0, ssem1):
    cid = lax.axis_index("c")
    sid = lax.axis_index("s")
    wid = cid * _NS + sid
    rows_per_w = stats_hbm.shape[0] // (_NC * _NS)

    @pl.when(sid == 0)
    def _zero():
        pltpu.sync_copy(zt_hbm, table)
        pltpu.sync_copy(zc_hbm, ctable)

    pltpu.sync_copy(ones_hbm, ones_v)
    pltpu.sync_copy(
        labels_hbm.at[pl.ds(wid * (rows_per_w // 128), rows_per_w // 128)],
        labels_v)
    plsc.subcore_barrier()

    base = wid * rows_per_w
    nchunk = rows_per_w // _G
    ksub = _G // _SCAT
    bufs = (sv0, sv1)
    lsems = (lsem0, lsem1)
    ssems = (ssem0, ssem1)

    def chunk_src(c):
        return stats_hbm.at[pl.ds(base + c * _G, _G)]

    loads = [pltpu.async_copy(chunk_src(0), bufs[0], lsems[0]), None]
    pending = [[], []]
    for c in range(nchunk):
        b = c % 2
        loads[b].wait()
        descs = []
        for k in range(ksub):
            idx = labels_v.at[c * ksub + k]
            descs.append(pltpu.async_copy(
                bufs[b].at[pl.ds(k * _SCAT, _SCAT)], table.at[idx],
                ssems[b], add=True))
            descs.append(pltpu.async_copy(
                ones_v, ctable.at[idx], ssems[b], add=True))
        if c + 1 < nchunk:
            nb = 1 - b
            for d in pending[nb]:
                d.wait()
            pending[nb] = []
            loads[nb] = pltpu.async_copy(chunk_src(c + 1), bufs[nb],
                                         lsems[nb])
        pending[b] = descs
    for b in (0, 1):
        for d in pending[b]:
            d.wait()

    plsc.subcore_barrier()

    @pl.when(sid == 0)
    def _emit():
        pltpu.sync_copy(table, s_out.at[cid])
        pltpu.sync_copy(ctable, c_out.at[cid])


def _fin_body(s_ref, c_ref, res_ref):
    sv = s_ref[0] + s_ref[1]
    cv = c_ref[0] + c_ref[1]
    counts = cv[:, 0:1]
    s = sv[:, _N_EFF:_D_EMB]
    s2 = sv[:, _D_EMB + _N_EFF:]
    n = jnp.maximum(counts, 1.0)
    m = s / n
    mse = (s2 - 2.0 * m * s) / n + m * m
    present = counts > 0.0
    n_present = jnp.maximum(jnp.sum(present.astype(jnp.float32)), 1.0)
    res = jnp.sum(jnp.where(present, mse, 0.0)) / (
        n_present * (_D_EMB - _N_EFF))
    res_ref[...] = jnp.full((8, 128), res, jnp.float32)


def kernel(x, W_enc, b_enc, W_dec, b_dec):
    b_rows = x.shape[0]
    grid = b_rows // _R
    we_pad = jnp.concatenate([jnp.zeros((1, _D_EMB), W_enc.dtype), W_enc],
                             axis=0)

    dec, stats, labels2d = pl.pallas_call(
        _enc_body,
        grid=(grid,),
        in_specs=[
            pl.BlockSpec((_R, _D_IN + 1), lambda i: (i, 0)),
            pl.BlockSpec((_D_IN + 1, _D_EMB), lambda i: (0, 0)),
            pl.BlockSpec((1, _D_EMB), lambda i: (0, 0)),
            pl.BlockSpec((_D_EMB, _D_IN), lambda i: (0, 0)),
            pl.BlockSpec((1, _D_IN), lambda i: (0, 0)),
        ],
        out_specs=[
            pl.BlockSpec((_R, _D_IN), lambda i: (i, 0)),
            pl.BlockSpec((_R, _SW), lambda i: (i, 0)),
            pl.BlockSpec((_R // 128, 128), lambda i: (i, 0)),
        ],
        out_shape=[
            jax.ShapeDtypeStruct((b_rows, _D_IN), jnp.float32),
            jax.ShapeDtypeStruct((b_rows, _SW), jnp.float32),
            jax.ShapeDtypeStruct((b_rows // 128, 128), jnp.int32),
        ],
    )(x, we_pad, b_enc.reshape(1, -1), W_dec, b_dec.reshape(1, -1))

    mesh = plsc.VectorSubcoreMesh(core_axis_name="c", subcore_axis_name="s")
    sc_fn = pl.kernel(
        _sc_body,
        out_type=[
            jax.ShapeDtypeStruct((_NC, _N_LABELS, _SW), jnp.float32),
            jax.ShapeDtypeStruct((_NC, _N_LABELS, 16), jnp.float32),
        ],
        mesh=mesh,
        scratch_types=[
            pltpu.VMEM_SHARED((_N_LABELS, _SW), jnp.float32),
            pltpu.VMEM_SHARED((_N_LABELS, 16), jnp.float32),
            pltpu.VMEM((_G, _SW), jnp.float32),
            pltpu.VMEM((_G, _SW), jnp.float32),
            pltpu.VMEM((64, 128), jnp.int32),
            pltpu.VMEM((_SCAT, 16), jnp.float32),
            pltpu.SemaphoreType.DMA,
            pltpu.SemaphoreType.DMA,
            pltpu.SemaphoreType.DMA,
            pltpu.SemaphoreType.DMA,
        ],
    )
    zt = jnp.zeros((_N_LABELS, _SW), jnp.float32)
    zc = jnp.zeros((_N_LABELS, 16), jnp.float32)
    ones = jnp.ones((_SCAT, 16), jnp.float32)
    s_tab, c_tab = sc_fn(stats, labels2d, zt, zc, ones)

    res = pl.pallas_call(
        _fin_body,
        out_shape=jax.ShapeDtypeStruct((8, 128), jnp.float32),
    )(s_tab, c_tab)
    return dec, res[0, 0]
